# baseline (speedup 1.0000x reference)
import functools

import jax
import jax.numpy as jnp
from jax import lax
from jax.experimental import pallas as pl
from jax.experimental.pallas import tpu as pltpu
from jax.experimental.pallas import tpu_sc as plsc

N = 10000
E = 320000
D_FEAT = 128
UNITS = 128
DIM = UNITS // 2
GAMMA = 1.0

NC = 2
NS = 16
LANES = 16
C = 80
EPT = E // NS
NBUF = 5
SB = 50
SEDGES = SB * C
NSUPER = EPT // SEDGES
ROWS_PT = N // NS

TC_BLOCK = 1000


def _tc_body(x_ref, w_ref, tab_ref, kl_ref):
    i = pl.program_id(0)
    h = jnp.dot(x_ref[...], w_ref[...], preferred_element_type=jnp.float32)
    m = h[:, :DIM]
    v = h[:, DIM:]
    mean = jnp.where(m > 0, m, jnp.exp(jnp.minimum(m, 0.0)) - 1.0)
    var = jnp.maximum(v, 0.0)
    att = jnp.exp(-GAMMA * var)

    def interleave(x):
        r = x.shape[0]
        h0 = jnp.stack([x[:, 0:16], x[:, 16:32]], axis=-1).reshape(r, 32)
        h1 = jnp.stack([x[:, 32:48], x[:, 48:64]], axis=-1).reshape(r, 32)
        return jnp.concatenate([h0, h1], axis=1)

    tab_ref[0] = interleave(mean * att).astype(jnp.bfloat16)
    tab_ref[1] = interleave(var * (att * att)).astype(jnp.bfloat16)
    kl_part = 0.5 * jnp.sum(
        jnp.mean(jnp.square(mean) + var - jnp.log(1e-8 + var) - 1.0, axis=1)
    )

    @pl.when(i == 0)
    def _():
        kl_ref[0, 0] = 0.0

    kl_ref[0, 0] += kl_part


def _tc_stage(features, w):
    return pl.pallas_call(
        _tc_body,
        grid=(N // TC_BLOCK,),
        in_specs=[
            pl.BlockSpec((TC_BLOCK, D_FEAT), lambda i: (i, 0)),
            pl.BlockSpec((D_FEAT, UNITS), lambda i: (0, 0)),
        ],
        out_specs=[
            pl.BlockSpec((2, TC_BLOCK, DIM), lambda i: (0, i, 0)),
            pl.BlockSpec(
                block_shape=(1, 1),
                index_map=lambda i: (0, 0),
                memory_space=pltpu.SMEM,
            ),
        ],
        out_shape=[
            jax.ShapeDtypeStruct((2, N, DIM), jnp.bfloat16),
            jax.ShapeDtypeStruct((1, 1), jnp.float32),
        ],
    )(features, w)


def _sc_kernel(tab_hbm, col_hbm, row_hbm, adj0_hbm, adj1_hbm, out_hbm,
               acc_sh, col_v, row_v, adj_v, rows_v, msg_v, gsems, ssems):
    cid = lax.axis_index("c")
    sid = lax.axis_index("s")

    zero = jnp.zeros((LANES,), jnp.float32)

    def zfill(r, carry):
        for f in range(DIM // LANES):
            msg_v[0, r, pl.ds(f * LANES, LANES)] = zero
        return carry

    lax.fori_loop(0, C, zfill, None, unroll=4)
    for k in range(ROWS_PT // C):
        pltpu.sync_copy(
            msg_v.at[0], acc_sh.at[pl.ds(sid * ROWS_PT + k * C, C)]
        )
    rem = ROWS_PT % C
    pltpu.sync_copy(
        msg_v.at[0].at[pl.ds(0, rem)],
        acc_sh.at[pl.ds(sid * ROWS_PT + (ROWS_PT // C) * C, rem)],
    )

    plsc.subcore_barrier()

    def start_gather(j, b):
        pltpu.async_copy(
            tab_hbm.at[col_v.at[pl.ds(j * C, C)]], rows_v.at[b], gsems.at[b]
        )

    def wait_gather(j, b):
        pltpu.make_async_copy(
            tab_hbm.at[col_v.at[pl.ds(j * C, C)]], rows_v.at[b], gsems.at[b]
        ).wait()

    def start_scatter(j, b):
        pltpu.async_copy(
            msg_v.at[b], acc_sh.at[row_v.at[pl.ds(j * C, C)]], ssems.at[b],
            add=True,
        )

    def wait_scatter(j, b):
        pltpu.make_async_copy(
            msg_v.at[b], acc_sh.at[row_v.at[pl.ds(j * C, C)]], ssems.at[b]
        ).wait()

    def scale_chunk(j, b):
        buf = rows_v.at[b]
        msg = msg_v.at[b]

        def ebody(e, carry):
            idx = jnp.full((LANES,), j * C + e, jnp.int32)
            a = plsc.load_gather(adj_v, [idx])
            for h in range(2):
                x = buf[e, pl.ds(32 * h, 32)]
                u0, u1 = plsc.unpack(x, format=plsc.PackFormat.INTERLEAVED)
                msg[e, pl.ds(32 * h, LANES)] = u0 * a
                msg[e, pl.ds(32 * h + LANES, LANES)] = u1 * a
            return carry

        lax.fori_loop(0, C, ebody, None, unroll=8)

    def super_body(s, carry):
        sbase = sid * EPT + s * SEDGES
        pltpu.sync_copy(col_hbm.at[pl.ds(sbase, SEDGES)], col_v)
        pltpu.sync_copy(row_hbm.at[pl.ds(sbase, SEDGES)], row_v)

        @pl.when(cid == 0)
        def _():
            pltpu.sync_copy(adj0_hbm.at[pl.ds(sbase, SEDGES)], adj_v)

        @pl.when(cid == 1)
        def _():
            pltpu.sync_copy(adj1_hbm.at[pl.ds(sbase, SEDGES)], adj_v)
            offs = jnp.full((LANES,), N, jnp.int32)

            def add_off(g, c2):
                sl = pl.ds(g * LANES, LANES)
                col_v[sl] = col_v[sl] + offs
                return c2

            lax.fori_loop(0, SEDGES // LANES, add_off, None, unroll=8)

        for b in range(NBUF - 1):
            start_gather(b, b)

        def body(i, c2):
            for b in range(NBUF):
                j = i * NBUF + b
                bprev = (b - 1) % NBUF
                wait_gather(j, b)
                scale_chunk(j, b)
                start_scatter(j, b)

                @pl.when(j >= 1)
                def _():
                    wait_scatter(j - 1, bprev)

                @pl.when(j + NBUF - 1 < SB)
                def _():
                    start_gather(j + NBUF - 1, bprev)
            return c2

        lax.fori_loop(0, SB // NBUF, body, None)
        wait_scatter(SB - 1, (SB - 1) % NBUF)
        return carry

    lax.fori_loop(0, NSUPER, super_body, None)

    plsc.subcore_barrier()
    pltpu.sync_copy(
        acc_sh.at[pl.ds(sid * ROWS_PT, ROWS_PT)],
        out_hbm.at[pl.ds(sid * ROWS_PT, ROWS_PT), pl.ds(cid * DIM, DIM)],
    )


def _sc_stage(tab2, col, row, adj0, adj1):
    mesh = plsc.VectorSubcoreMesh(core_axis_name="c", subcore_axis_name="s")
    run = functools.partial(
        pl.kernel,
        out_type=jax.ShapeDtypeStruct((N, UNITS), jnp.float32),
        mesh=mesh,
        scratch_types=[
            pltpu.VMEM_SHARED((N, DIM), jnp.float32),
            pltpu.VMEM((SEDGES,), jnp.int32),
            pltpu.VMEM((SEDGES,), jnp.int32),
            pltpu.VMEM((SEDGES,), jnp.float32),
            pltpu.VMEM((NBUF, C, DIM), jnp.bfloat16),
            pltpu.VMEM((NBUF, C, DIM), jnp.float32),
            pltpu.SemaphoreType.DMA((NBUF,)),
            pltpu.SemaphoreType.DMA((NBUF,)),
        ],
        compiler_params=pltpu.CompilerParams(
            use_tc_tiling_on_sc=False, needs_layout_passes=False
        ),
    )(_sc_kernel)
    return run(tab2, col, row, adj0, adj1)


def kernel(features, edge_index, adj0_vals, adj1_vals, kernel):
    tab, kl = _tc_stage(features, kernel)
    output = _sc_stage(
        tab.reshape(2 * N, DIM),
        edge_index[1], edge_index[0], adj0_vals, adj1_vals,
    )
    return (output, kl[0, 0])


# speedup vs baseline: 2.3850x; 2.3850x over previous
import functools

import jax
import jax.numpy as jnp
from jax import lax
from jax.experimental import pallas as pl
from jax.experimental.pallas import tpu as pltpu
from jax.experimental.pallas import tpu_sc as plsc

N = 10000
E = 320000
D_FEAT = 128
UNITS = 128
DIM = UNITS // 2
GAMMA = 1.0

NC = 2
NS = 16
LANES = 16
C = 80
EPT = E // NS
NBUF = 10
SB = 50
SEDGES = SB * C
NSUPER = EPT // SEDGES
ROWS_PT = N // NS

TC_BLOCK = 1000


def _tc_body(x_ref, w_ref, tab_ref, kl_ref):
    i = pl.program_id(0)
    h = jnp.dot(x_ref[...], w_ref[...], preferred_element_type=jnp.float32)
    m = h[:, :DIM]
    v = h[:, DIM:]
    mean = jnp.where(m > 0, m, jnp.exp(jnp.minimum(m, 0.0)) - 1.0)
    var = jnp.maximum(v, 0.0)
    att = jnp.exp(-GAMMA * var)
    tab_ref[0] = mean * att
    tab_ref[1] = var * (att * att)
    kl_part = 0.5 * jnp.sum(
        jnp.mean(jnp.square(mean) + var - jnp.log(1e-8 + var) - 1.0, axis=1)
    )

    @pl.when(i == 0)
    def _():
        kl_ref[0, 0] = 0.0

    kl_ref[0, 0] += kl_part


def _tc_stage(features, w):
    return pl.pallas_call(
        _tc_body,
        grid=(N // TC_BLOCK,),
        in_specs=[
            pl.BlockSpec((TC_BLOCK, D_FEAT), lambda i: (i, 0)),
            pl.BlockSpec((D_FEAT, UNITS), lambda i: (0, 0)),
        ],
        out_specs=[
            pl.BlockSpec((2, TC_BLOCK, DIM), lambda i: (0, i, 0)),
            pl.BlockSpec(
                block_shape=(1, 1),
                index_map=lambda i: (0, 0),
                memory_space=pltpu.SMEM,
            ),
        ],
        out_shape=[
            jax.ShapeDtypeStruct((2, N, DIM), jnp.float32),
            jax.ShapeDtypeStruct((1, 1), jnp.float32),
        ],
    )(features, w)


def _sc_kernel(tab_hbm, ei_hbm, adj0_hbm, adj1_hbm, out_hbm,
               acc_sh, col_v, row_v, adj_v, rows_v, gsems, ssems):
    cid = lax.axis_index("c")
    sid = lax.axis_index("s")

    zero = jnp.zeros((LANES,), jnp.float32)

    def zfill(r, carry):
        for f in range(DIM // LANES):
            rows_v[0, r, pl.ds(f * LANES, LANES)] = zero
        return carry

    lax.fori_loop(0, C, zfill, None, unroll=4)
    for k in range(ROWS_PT // C):
        pltpu.sync_copy(
            rows_v.at[0], acc_sh.at[pl.ds(sid * ROWS_PT + k * C, C)]
        )
    rem = ROWS_PT % C
    pltpu.sync_copy(
        rows_v.at[0].at[pl.ds(0, rem)],
        acc_sh.at[pl.ds(sid * ROWS_PT + (ROWS_PT // C) * C, rem)],
    )

    plsc.subcore_barrier()

    def start_gather(j, b):
        pltpu.async_copy(
            tab_hbm.at[col_v.at[pl.ds(j * C, C)]], rows_v.at[b], gsems.at[b]
        )

    def wait_gather(j, b):
        pltpu.make_async_copy(
            tab_hbm.at[col_v.at[pl.ds(j * C, C)]], rows_v.at[b], gsems.at[b]
        ).wait()

    def start_scatter(j, b):
        pltpu.async_copy(
            rows_v.at[b], acc_sh.at[row_v.at[pl.ds(j * C, C)]], ssems.at[b],
            add=True,
        )

    def wait_scatter(j, b):
        pltpu.make_async_copy(
            rows_v.at[b], acc_sh.at[row_v.at[pl.ds(j * C, C)]], ssems.at[b]
        ).wait()

    def scale_chunk(j, b):
        buf = rows_v.at[b]

        def ebody(e, carry):
            idx = jnp.full((LANES,), j * C + e, jnp.int32)
            a = plsc.load_gather(adj_v, [idx])
            for f in range(DIM // LANES):
                sl = pl.ds(f * LANES, LANES)
                buf[e, sl] = buf[e, sl] * a
            return carry

        lax.fori_loop(0, C, ebody, None, unroll=8)

    def super_body(s, carry):
        sbase = sid * EPT + s * SEDGES
        pltpu.sync_copy(ei_hbm.at[1, pl.ds(sbase, SEDGES)], col_v)
        pltpu.sync_copy(ei_hbm.at[0, pl.ds(sbase, SEDGES)], row_v)

        @pl.when(cid == 0)
        def _():
            pltpu.sync_copy(adj0_hbm.at[pl.ds(sbase, SEDGES)], adj_v)

        @pl.when(cid == 1)
        def _():
            pltpu.sync_copy(adj1_hbm.at[pl.ds(sbase, SEDGES)], adj_v)
            offs = jnp.full((LANES,), N, jnp.int32)

            def add_off(g, c2):
                sl = pl.ds(g * LANES, LANES)
                col_v[sl] = col_v[sl] + offs
                return c2

            lax.fori_loop(0, SEDGES // LANES, add_off, None, unroll=8)

        for b in range(NBUF - 1):
            start_gather(b, b)

        def body(i, c2):
            for b in range(NBUF):
                j = i * NBUF + b
                bprev = (b - 1) % NBUF
                wait_gather(j, b)
                scale_chunk(j, b)
                start_scatter(j, b)

                @pl.when(j >= 1)
                def _():
                    wait_scatter(j - 1, bprev)

                @pl.when(j + NBUF - 1 < SB)
                def _():
                    start_gather(j + NBUF - 1, bprev)
            return c2

        lax.fori_loop(0, SB // NBUF, body, None)
        wait_scatter(SB - 1, (SB - 1) % NBUF)
        return carry

    lax.fori_loop(0, NSUPER, super_body, None)

    plsc.subcore_barrier()
    pltpu.sync_copy(
        acc_sh.at[pl.ds(sid * ROWS_PT, ROWS_PT)],
        out_hbm.at[pl.ds(sid * ROWS_PT, ROWS_PT), pl.ds(cid * DIM, DIM)],
    )


def _sc_stage(tab2, ei, adj0, adj1):
    mesh = plsc.VectorSubcoreMesh(core_axis_name="c", subcore_axis_name="s")
    run = functools.partial(
        pl.kernel,
        out_type=jax.ShapeDtypeStruct((N, UNITS), jnp.float32),
        mesh=mesh,
        scratch_types=[
            pltpu.VMEM_SHARED((N, DIM), jnp.float32),
            pltpu.VMEM((SEDGES,), jnp.int32),
            pltpu.VMEM((SEDGES,), jnp.int32),
            pltpu.VMEM((SEDGES,), jnp.float32),
            pltpu.VMEM((NBUF, C, DIM), jnp.float32),
            pltpu.SemaphoreType.DMA((NBUF,)),
            pltpu.SemaphoreType.DMA((NBUF,)),
        ],
        compiler_params=pltpu.CompilerParams(
            use_tc_tiling_on_sc=False, needs_layout_passes=False
        ),
    )(_sc_kernel)
    return run(tab2, ei, adj0, adj1)


def kernel(features, edge_index, adj0_vals, adj1_vals, kernel):
    tab, kl = _tc_stage(features, kernel)
    output = _sc_stage(
        tab.reshape(2 * N, DIM), edge_index, adj0_vals, adj1_vals
    )
    return (output, kl[0, 0])


# speedup vs baseline: 2.5700x; 1.0776x over previous
import functools

import jax
import jax.numpy as jnp
from jax import lax
from jax.experimental import pallas as pl
from jax.experimental.pallas import tpu as pltpu
from jax.experimental.pallas import tpu_sc as plsc

N = 10000
E = 320000
D_FEAT = 128
UNITS = 128
DIM = UNITS // 2
GAMMA = 1.0

NC = 2
NS = 16
LANES = 16
C = 80
EPT = E // NS
NBUF = 5
SB = 50
SEDGES = SB * C
NSUPER = EPT // SEDGES
ROWS_PT = N // NS

TC_BLOCK = 1000


def _tc_body(x_ref, w_ref, tab_ref, kl_ref):
    i = pl.program_id(0)
    h = jnp.dot(x_ref[...], w_ref[...], preferred_element_type=jnp.float32)
    m = h[:, :DIM]
    v = h[:, DIM:]
    mean = jnp.where(m > 0, m, jnp.exp(jnp.minimum(m, 0.0)) - 1.0)
    var = jnp.maximum(v, 0.0)
    att = jnp.exp(-GAMMA * var)
    tab_ref[0] = mean * att
    tab_ref[1] = var * (att * att)
    kl_part = 0.5 * jnp.sum(
        jnp.mean(jnp.square(mean) + var - jnp.log(1e-8 + var) - 1.0, axis=1)
    )

    @pl.when(i == 0)
    def _():
        kl_ref[0, 0] = 0.0

    kl_ref[0, 0] += kl_part


def _tc_stage(features, w):
    return pl.pallas_call(
        _tc_body,
        grid=(N // TC_BLOCK,),
        in_specs=[
            pl.BlockSpec((TC_BLOCK, D_FEAT), lambda i: (i, 0)),
            pl.BlockSpec((D_FEAT, UNITS), lambda i: (0, 0)),
        ],
        out_specs=[
            pl.BlockSpec((2, TC_BLOCK, DIM), lambda i: (0, i, 0)),
            pl.BlockSpec(
                block_shape=(1, 1),
                index_map=lambda i: (0, 0),
                memory_space=pltpu.SMEM,
            ),
        ],
        out_shape=[
            jax.ShapeDtypeStruct((2, N, DIM), jnp.float32),
            jax.ShapeDtypeStruct((1, 1), jnp.float32),
        ],
    )(features, w)


def _sc_kernel(tab_hbm, ei_hbm, adj0_hbm, adj1_hbm, out_hbm,
               acc_sh, col_v, row_v, adj_v, rows_v, gsems, ssems):
    cid = lax.axis_index("c")
    sid = lax.axis_index("s")

    zero = jnp.zeros((LANES,), jnp.float32)

    def zfill(r, carry):
        for f in range(DIM // LANES):
            rows_v[0, r, pl.ds(f * LANES, LANES)] = zero
        return carry

    lax.fori_loop(0, C, zfill, None, unroll=4)
    for k in range(ROWS_PT // C):
        pltpu.sync_copy(
            rows_v.at[0], acc_sh.at[pl.ds(sid * ROWS_PT + k * C, C)]
        )
    rem = ROWS_PT % C
    pltpu.sync_copy(
        rows_v.at[0].at[pl.ds(0, rem)],
        acc_sh.at[pl.ds(sid * ROWS_PT + (ROWS_PT // C) * C, rem)],
    )

    plsc.subcore_barrier()

    def start_gather(j, b):
        pltpu.async_copy(
            tab_hbm.at[col_v.at[pl.ds(j * C, C)]], rows_v.at[b], gsems.at[b]
        )

    def wait_gather(j, b):
        pltpu.make_async_copy(
            tab_hbm.at[col_v.at[pl.ds(j * C, C)]], rows_v.at[b], gsems.at[b]
        ).wait()

    def start_scatter(j, b):
        pltpu.async_copy(
            rows_v.at[b], acc_sh.at[row_v.at[pl.ds(j * C, C)]], ssems.at[b],
            add=True,
        )

    def wait_scatter(j, b):
        pltpu.make_async_copy(
            rows_v.at[b], acc_sh.at[row_v.at[pl.ds(j * C, C)]], ssems.at[b]
        ).wait()

    def scale_chunk(j, b):
        buf = rows_v.at[b]

        def ebody(e, carry):
            idx = jnp.full((LANES,), j * C + e, jnp.int32)
            a = plsc.load_gather(adj_v, [idx])
            for f in range(DIM // LANES):
                sl = pl.ds(f * LANES, LANES)
                buf[e, sl] = buf[e, sl] * a
            return carry

        lax.fori_loop(0, C, ebody, None, unroll=8)

    def super_body(s, carry):
        sbase = sid * EPT + s * SEDGES
        pltpu.sync_copy(ei_hbm.at[1, pl.ds(sbase, SEDGES)], col_v)
        pltpu.sync_copy(ei_hbm.at[0, pl.ds(sbase, SEDGES)], row_v)

        @pl.when(cid == 0)
        def _():
            pltpu.sync_copy(adj0_hbm.at[pl.ds(sbase, SEDGES)], adj_v)

        @pl.when(cid == 1)
        def _():
            pltpu.sync_copy(adj1_hbm.at[pl.ds(sbase, SEDGES)], adj_v)
            offs = jnp.full((LANES,), N, jnp.int32)

            def add_off(g, c2):
                sl = pl.ds(g * LANES, LANES)
                col_v[sl] = col_v[sl] + offs
                return c2

            lax.fori_loop(0, SEDGES // LANES, add_off, None, unroll=8)

        for b in range(NBUF - 1):
            start_gather(b, b)

        def body(i, c2):
            for b in range(NBUF):
                j = i * NBUF + b
                bprev = (b - 1) % NBUF
                wait_gather(j, b)
                scale_chunk(j, b)
                start_scatter(j, b)

                @pl.when(j >= 1)
                def _():
                    wait_scatter(j - 1, bprev)

                @pl.when(j + NBUF - 1 < SB)
                def _():
                    start_gather(j + NBUF - 1, bprev)
            return c2

        lax.fori_loop(0, SB // NBUF, body, None)
        wait_scatter(SB - 1, (SB - 1) % NBUF)
        return carry

    lax.fori_loop(0, NSUPER, super_body, None)

    plsc.subcore_barrier()
    pltpu.sync_copy(
        acc_sh.at[pl.ds(sid * ROWS_PT, ROWS_PT)],
        out_hbm.at[pl.ds(sid * ROWS_PT, ROWS_PT), pl.ds(cid * DIM, DIM)],
    )


def _sc_stage(tab2, ei, adj0, adj1):
    mesh = plsc.VectorSubcoreMesh(core_axis_name="c", subcore_axis_name="s")
    run = functools.partial(
        pl.kernel,
        out_type=jax.ShapeDtypeStruct((N, UNITS), jnp.float32),
        mesh=mesh,
        scratch_types=[
            pltpu.VMEM_SHARED((N, DIM), jnp.float32),
            pltpu.VMEM((SEDGES,), jnp.int32),
            pltpu.VMEM((SEDGES,), jnp.int32),
            pltpu.VMEM((SEDGES,), jnp.float32),
            pltpu.VMEM((NBUF, C, DIM), jnp.float32),
            pltpu.SemaphoreType.DMA((NBUF,)),
            pltpu.SemaphoreType.DMA((NBUF,)),
        ],
        compiler_params=pltpu.CompilerParams(
            use_tc_tiling_on_sc=False, needs_layout_passes=False
        ),
    )(_sc_kernel)
    return run(tab2, ei, adj0, adj1)


def kernel(features, edge_index, adj0_vals, adj1_vals, kernel):
    tab, kl = _tc_stage(features, kernel)
    output = _sc_stage(
        tab.reshape(2 * N, DIM), edge_index, adj0_vals, adj1_vals
    )
    return (output, kl[0, 0])


# speedup vs baseline: 3.0902x; 1.2024x over previous
import functools

import jax
import jax.numpy as jnp
from jax import lax
from jax.experimental import pallas as pl
from jax.experimental.pallas import tpu as pltpu
from jax.experimental.pallas import tpu_sc as plsc

N = 10000
E = 320000
D_FEAT = 128
UNITS = 128
DIM = UNITS // 2
GAMMA = 1.0

NC = 2
NS = 16
NW = NC * NS
LANES = 16
C = 40
EPW = E // NW
NBUF = 5
SB = 50
SEDGES = SB * C
NSUPER = EPW // SEDGES
ROWS_PT = N // NS

TC_BLOCK = 1000


def _tc_body(x_ref, w_ref, tab_ref, kl_ref):
    i = pl.program_id(0)
    h = jnp.dot(x_ref[...], w_ref[...], preferred_element_type=jnp.float32)
    m = h[:, :DIM]
    v = h[:, DIM:]
    mean = jnp.where(m > 0, m, jnp.exp(jnp.minimum(m, 0.0)) - 1.0)
    var = jnp.maximum(v, 0.0)
    att = jnp.exp(-GAMMA * var)
    tab_ref[:, :DIM] = mean * att
    tab_ref[:, DIM:] = var * (att * att)
    kl_part = 0.5 * jnp.sum(
        jnp.mean(jnp.square(mean) + var - jnp.log(1e-8 + var) - 1.0, axis=1)
    )

    @pl.when(i == 0)
    def _():
        kl_ref[0, 0] = 0.0

    kl_ref[0, 0] += kl_part


def _tc_stage(features, w):
    return pl.pallas_call(
        _tc_body,
        grid=(N // TC_BLOCK,),
        in_specs=[
            pl.BlockSpec((TC_BLOCK, D_FEAT), lambda i: (i, 0)),
            pl.BlockSpec((D_FEAT, UNITS), lambda i: (0, 0)),
        ],
        out_specs=[
            pl.BlockSpec((TC_BLOCK, UNITS), lambda i: (i, 0)),
            pl.BlockSpec(
                block_shape=(1, 1),
                index_map=lambda i: (0, 0),
                memory_space=pltpu.SMEM,
            ),
        ],
        out_shape=[
            jax.ShapeDtypeStruct((N, UNITS), jnp.float32),
            jax.ShapeDtypeStruct((1, 1), jnp.float32),
        ],
    )(features, w)


def _merge_body(p_ref, out_ref):
    out_ref[...] = p_ref[0] + p_ref[1]


def _merge_stage(partials):
    return pl.pallas_call(
        _merge_body,
        grid=(N // TC_BLOCK,),
        in_specs=[pl.BlockSpec((2, TC_BLOCK, UNITS), lambda i: (0, i, 0))],
        out_specs=pl.BlockSpec((TC_BLOCK, UNITS), lambda i: (i, 0)),
        out_shape=jax.ShapeDtypeStruct((N, UNITS), jnp.float32),
    )(partials)


def _sc_kernel(tab_hbm, ei_hbm, adj0_hbm, adj1_hbm, out_hbm,
               acc_sh, col_v, row_v, adj0_v, adj1_v, rows_v, gsems, ssems):
    cid = lax.axis_index("c")
    sid = lax.axis_index("s")
    wid = cid * NS + sid

    zero = jnp.zeros((LANES,), jnp.float32)

    def zfill(r, carry):
        for f in range(UNITS // LANES):
            rows_v[0, r, pl.ds(f * LANES, LANES)] = zero
        return carry

    lax.fori_loop(0, C, zfill, None, unroll=4)
    for k in range(ROWS_PT // C):
        pltpu.sync_copy(
            rows_v.at[0], acc_sh.at[pl.ds(sid * ROWS_PT + k * C, C)]
        )
    rem = ROWS_PT % C
    pltpu.sync_copy(
        rows_v.at[0].at[pl.ds(0, rem)],
        acc_sh.at[pl.ds(sid * ROWS_PT + (ROWS_PT // C) * C, rem)],
    )

    plsc.subcore_barrier()

    def start_gather(j, b):
        pltpu.async_copy(
            tab_hbm.at[col_v.at[pl.ds(j * C, C)]], rows_v.at[b], gsems.at[b]
        )

    def wait_gather(j, b):
        pltpu.make_async_copy(
            tab_hbm.at[col_v.at[pl.ds(j * C, C)]], rows_v.at[b], gsems.at[b]
        ).wait()

    def start_scatter(j, b):
        pltpu.async_copy(
            rows_v.at[b], acc_sh.at[row_v.at[pl.ds(j * C, C)]], ssems.at[b],
            add=True,
        )

    def wait_scatter(j, b):
        pltpu.make_async_copy(
            rows_v.at[b], acc_sh.at[row_v.at[pl.ds(j * C, C)]], ssems.at[b]
        ).wait()

    def scale_chunk(j, b):
        buf = rows_v.at[b]

        def ebody(e, carry):
            idx = jnp.full((LANES,), j * C + e, jnp.int32)
            a0 = plsc.load_gather(adj0_v, [idx])
            a1 = plsc.load_gather(adj1_v, [idx])
            for f in range(DIM // LANES):
                sl = pl.ds(f * LANES, LANES)
                buf[e, sl] = buf[e, sl] * a0
            for f in range(DIM // LANES, UNITS // LANES):
                sl = pl.ds(f * LANES, LANES)
                buf[e, sl] = buf[e, sl] * a1
            return carry

        lax.fori_loop(0, C, ebody, None, unroll=4)

    def super_body(s, carry):
        sbase = wid * EPW + s * SEDGES
        pltpu.sync_copy(ei_hbm.at[1, pl.ds(sbase, SEDGES)], col_v)
        pltpu.sync_copy(ei_hbm.at[0, pl.ds(sbase, SEDGES)], row_v)
        pltpu.sync_copy(adj0_hbm.at[pl.ds(sbase, SEDGES)], adj0_v)
        pltpu.sync_copy(adj1_hbm.at[pl.ds(sbase, SEDGES)], adj1_v)

        for b in range(NBUF - 1):
            start_gather(b, b)

        def body(i, c2):
            for b in range(NBUF):
                j = i * NBUF + b
                bprev = (b - 1) % NBUF
                wait_gather(j, b)
                scale_chunk(j, b)
                start_scatter(j, b)

                @pl.when(j >= 1)
                def _():
                    wait_scatter(j - 1, bprev)

                @pl.when(j + NBUF - 1 < SB)
                def _():
                    start_gather(j + NBUF - 1, bprev)
            return c2

        lax.fori_loop(0, SB // NBUF, body, None)
        wait_scatter(SB - 1, (SB - 1) % NBUF)
        return carry

    lax.fori_loop(0, NSUPER, super_body, None)

    plsc.subcore_barrier()
    pltpu.sync_copy(
        acc_sh.at[pl.ds(sid * ROWS_PT, ROWS_PT)],
        out_hbm.at[cid, pl.ds(sid * ROWS_PT, ROWS_PT)],
    )


def _sc_stage(tab, ei, adj0, adj1):
    mesh = plsc.VectorSubcoreMesh(core_axis_name="c", subcore_axis_name="s")
    run = functools.partial(
        pl.kernel,
        out_type=jax.ShapeDtypeStruct((NC, N, UNITS), jnp.float32),
        mesh=mesh,
        scratch_types=[
            pltpu.VMEM_SHARED((N, UNITS), jnp.float32),
            pltpu.VMEM((SEDGES,), jnp.int32),
            pltpu.VMEM((SEDGES,), jnp.int32),
            pltpu.VMEM((SEDGES,), jnp.float32),
            pltpu.VMEM((SEDGES,), jnp.float32),
            pltpu.VMEM((NBUF, C, UNITS), jnp.float32),
            pltpu.SemaphoreType.DMA((NBUF,)),
            pltpu.SemaphoreType.DMA((NBUF,)),
        ],
        compiler_params=pltpu.CompilerParams(
            use_tc_tiling_on_sc=False, needs_layout_passes=False
        ),
    )(_sc_kernel)
    return run(tab, ei, adj0, adj1)


def kernel(features, edge_index, adj0_vals, adj1_vals, kernel):
    tab, kl = _tc_stage(features, kernel)
    partials = _sc_stage(tab, edge_index, adj0_vals, adj1_vals)
    output = _merge_stage(partials)
    return (output, kl[0, 0])


# speedup vs baseline: 3.2572x; 1.0540x over previous
import functools

import jax
import jax.numpy as jnp
from jax import lax
from jax.experimental import pallas as pl
from jax.experimental.pallas import tpu as pltpu
from jax.experimental.pallas import tpu_sc as plsc

N = 10000
E = 320000
D_FEAT = 128
UNITS = 128
DIM = UNITS // 2
GAMMA = 1.0

NC = 2
NS = 16
NW = NC * NS
LANES = 16
C = 40
EPW = E // NW
NBUF = 5
SB = 50
SEDGES = SB * C
NSUPER = EPW // SEDGES
ROWS_PT = N // NS

TC_BLOCK = 1000


def _tc_body(x_ref, w_ref, tab_ref, kl_ref):
    i = pl.program_id(0)
    h = jnp.dot(x_ref[...], w_ref[...], preferred_element_type=jnp.float32)
    m = h[:, :DIM]
    v = h[:, DIM:]
    mean = jnp.where(m > 0, m, jnp.exp(jnp.minimum(m, 0.0)) - 1.0)
    var = jnp.maximum(v, 0.0)
    att = jnp.exp(-GAMMA * var)
    tab_ref[:, :DIM] = mean * att
    tab_ref[:, DIM:] = var * (att * att)
    kl_part = 0.5 * jnp.sum(
        jnp.mean(jnp.square(mean) + var - jnp.log(1e-8 + var) - 1.0, axis=1)
    )

    @pl.when(i == 0)
    def _():
        kl_ref[0, 0] = 0.0

    kl_ref[0, 0] += kl_part


def _tc_stage(features, w):
    return pl.pallas_call(
        _tc_body,
        grid=(N // TC_BLOCK,),
        in_specs=[
            pl.BlockSpec((TC_BLOCK, D_FEAT), lambda i: (i, 0)),
            pl.BlockSpec((D_FEAT, UNITS), lambda i: (0, 0)),
        ],
        out_specs=[
            pl.BlockSpec((TC_BLOCK, UNITS), lambda i: (i, 0)),
            pl.BlockSpec(
                block_shape=(1, 1),
                index_map=lambda i: (0, 0),
                memory_space=pltpu.SMEM,
            ),
        ],
        out_shape=[
            jax.ShapeDtypeStruct((N, UNITS), jnp.float32),
            jax.ShapeDtypeStruct((1, 1), jnp.float32),
        ],
    )(features, w)


def _merge_body(p_ref, out_ref):
    out_ref[...] = p_ref[0] + p_ref[1]


def _merge_stage(partials):
    return pl.pallas_call(
        _merge_body,
        grid=(N // TC_BLOCK,),
        in_specs=[pl.BlockSpec((2, TC_BLOCK, UNITS), lambda i: (0, i, 0))],
        out_specs=pl.BlockSpec((TC_BLOCK, UNITS), lambda i: (i, 0)),
        out_shape=jax.ShapeDtypeStruct((N, UNITS), jnp.float32),
    )(partials)


def _sc_kernel(tab_hbm, ei_hbm, adj0_hbm, adj1_hbm, out_hbm,
               acc_sh, col_v, row_v, adj0_v, adj1_v, rows_v, gsems, ssems):
    cid = lax.axis_index("c")
    sid = lax.axis_index("s")
    wid = cid * NS + sid

    zero = jnp.zeros((LANES,), jnp.float32)

    def zfill(r, carry):
        for f in range(UNITS // LANES):
            rows_v[0, r, pl.ds(f * LANES, LANES)] = zero
        return carry

    lax.fori_loop(0, C, zfill, None, unroll=4)
    rem = ROWS_PT % C
    zdescs = []
    for k in range(ROWS_PT // C):
        zdescs.append(pltpu.make_async_copy(
            rows_v.at[0], acc_sh.at[pl.ds(sid * ROWS_PT + k * C, C)],
            ssems.at[0],
        ))
    zdescs.append(pltpu.make_async_copy(
        rows_v.at[0].at[pl.ds(0, rem)],
        acc_sh.at[pl.ds(sid * ROWS_PT + (ROWS_PT // C) * C, rem)],
        ssems.at[0],
    ))
    for d in zdescs:
        d.start()
    for d in zdescs:
        d.wait()

    plsc.subcore_barrier()

    def start_gather(j, b):
        pltpu.async_copy(
            tab_hbm.at[col_v.at[pl.ds(j * C, C)]], rows_v.at[b], gsems.at[b]
        )

    def wait_gather(j, b):
        pltpu.make_async_copy(
            tab_hbm.at[col_v.at[pl.ds(j * C, C)]], rows_v.at[b], gsems.at[b]
        ).wait()

    def start_scatter(j, b):
        pltpu.async_copy(
            rows_v.at[b], acc_sh.at[row_v.at[pl.ds(j * C, C)]], ssems.at[b],
            add=True,
        )

    def wait_scatter(j, b):
        pltpu.make_async_copy(
            rows_v.at[b], acc_sh.at[row_v.at[pl.ds(j * C, C)]], ssems.at[b]
        ).wait()

    def scale_chunk(j, b):
        buf = rows_v.at[b]

        def ebody(e, carry):
            idx = jnp.full((LANES,), j * C + e, jnp.int32)
            a0 = plsc.load_gather(adj0_v, [idx])
            a1 = plsc.load_gather(adj1_v, [idx])
            for f in range(DIM // LANES):
                sl = pl.ds(f * LANES, LANES)
                buf[e, sl] = buf[e, sl] * a0
            for f in range(DIM // LANES, UNITS // LANES):
                sl = pl.ds(f * LANES, LANES)
                buf[e, sl] = buf[e, sl] * a1
            return carry

        lax.fori_loop(0, C, ebody, None, unroll=4)

    def super_body(s, carry):
        sbase = wid * EPW + s * SEDGES
        sdescs = [
            pltpu.make_async_copy(
                ei_hbm.at[1, pl.ds(sbase, SEDGES)], col_v, gsems.at[0]),
            pltpu.make_async_copy(
                ei_hbm.at[0, pl.ds(sbase, SEDGES)], row_v, gsems.at[0]),
            pltpu.make_async_copy(
                adj0_hbm.at[pl.ds(sbase, SEDGES)], adj0_v, gsems.at[0]),
            pltpu.make_async_copy(
                adj1_hbm.at[pl.ds(sbase, SEDGES)], adj1_v, gsems.at[0]),
        ]
        for d in sdescs:
            d.start()
        for d in sdescs:
            d.wait()

        for b in range(NBUF - 1):
            start_gather(b, b)

        def body(i, c2):
            for b in range(NBUF):
                j = i * NBUF + b
                bprev = (b - 1) % NBUF
                wait_gather(j, b)
                scale_chunk(j, b)
                start_scatter(j, b)

                @pl.when(j >= 1)
                def _():
                    wait_scatter(j - 1, bprev)

                @pl.when(j + NBUF - 1 < SB)
                def _():
                    start_gather(j + NBUF - 1, bprev)
            return c2

        lax.fori_loop(0, SB // NBUF, body, None)
        wait_scatter(SB - 1, (SB - 1) % NBUF)
        return carry

    lax.fori_loop(0, NSUPER, super_body, None)

    plsc.subcore_barrier()
    pltpu.sync_copy(
        acc_sh.at[pl.ds(sid * ROWS_PT, ROWS_PT)],
        out_hbm.at[cid, pl.ds(sid * ROWS_PT, ROWS_PT)],
    )


def _sc_stage(tab, ei, adj0, adj1):
    mesh = plsc.VectorSubcoreMesh(core_axis_name="c", subcore_axis_name="s")
    run = functools.partial(
        pl.kernel,
        out_type=jax.ShapeDtypeStruct((NC, N, UNITS), jnp.float32),
        mesh=mesh,
        scratch_types=[
            pltpu.VMEM_SHARED((N, UNITS), jnp.float32),
            pltpu.VMEM((SEDGES,), jnp.int32),
            pltpu.VMEM((SEDGES,), jnp.int32),
            pltpu.VMEM((SEDGES,), jnp.float32),
            pltpu.VMEM((SEDGES,), jnp.float32),
            pltpu.VMEM((NBUF, C, UNITS), jnp.float32),
            pltpu.SemaphoreType.DMA((NBUF,)),
            pltpu.SemaphoreType.DMA((NBUF,)),
        ],
        compiler_params=pltpu.CompilerParams(
            use_tc_tiling_on_sc=False, needs_layout_passes=False
        ),
    )(_sc_kernel)
    return run(tab, ei, adj0, adj1)


def kernel(features, edge_index, adj0_vals, adj1_vals, kernel):
    tab, kl = _tc_stage(features, kernel)
    partials = _sc_stage(tab, edge_index, adj0_vals, adj1_vals)
    output = _merge_stage(partials)
    return (output, kl[0, 0])


# speedup vs baseline: 3.3011x; 1.0135x over previous
import functools

import jax
import jax.numpy as jnp
from jax import lax
from jax.experimental import pallas as pl
from jax.experimental.pallas import tpu as pltpu
from jax.experimental.pallas import tpu_sc as plsc

N = 10000
E = 320000
D_FEAT = 128
UNITS = 128
DIM = UNITS // 2
GAMMA = 1.0

NC = 2
NS = 16
NW = NC * NS
LANES = 16
C = 40
EPW = E // NW
NBUF = 5
SB = 50
SEDGES = SB * C
NSUPER = EPW // SEDGES
ROWS_PT = N // NS

TC_BLOCK = 2000


def _tc_body(x_ref, w_ref, tab_ref, kl_ref):
    i = pl.program_id(0)
    h = jnp.dot(x_ref[...], w_ref[...], preferred_element_type=jnp.float32)
    m = h[:, :DIM]
    v = h[:, DIM:]
    mean = jnp.where(m > 0, m, jnp.exp(jnp.minimum(m, 0.0)) - 1.0)
    var = jnp.maximum(v, 0.0)
    att = jnp.exp(-GAMMA * var)
    tab_ref[:, :DIM] = mean * att
    tab_ref[:, DIM:] = var * (att * att)
    kl_part = 0.5 * jnp.sum(
        jnp.mean(jnp.square(mean) + var - jnp.log(1e-8 + var) - 1.0, axis=1)
    )

    @pl.when(i == 0)
    def _():
        kl_ref[0, 0] = 0.0

    kl_ref[0, 0] += kl_part


def _tc_stage(features, w):
    return pl.pallas_call(
        _tc_body,
        grid=(N // TC_BLOCK,),
        in_specs=[
            pl.BlockSpec((TC_BLOCK, D_FEAT), lambda i: (i, 0)),
            pl.BlockSpec((D_FEAT, UNITS), lambda i: (0, 0)),
        ],
        out_specs=[
            pl.BlockSpec((TC_BLOCK, UNITS), lambda i: (i, 0)),
            pl.BlockSpec(
                block_shape=(1, 1),
                index_map=lambda i: (0, 0),
                memory_space=pltpu.SMEM,
            ),
        ],
        out_shape=[
            jax.ShapeDtypeStruct((N, UNITS), jnp.float32),
            jax.ShapeDtypeStruct((1, 1), jnp.float32),
        ],
    )(features, w)


def _merge_body(p_ref, out_ref):
    out_ref[...] = p_ref[0] + p_ref[1]


def _merge_stage(partials):
    return pl.pallas_call(
        _merge_body,
        grid=(N // TC_BLOCK,),
        in_specs=[pl.BlockSpec((2, TC_BLOCK, UNITS), lambda i: (0, i, 0))],
        out_specs=pl.BlockSpec((TC_BLOCK, UNITS), lambda i: (i, 0)),
        out_shape=jax.ShapeDtypeStruct((N, UNITS), jnp.float32),
    )(partials)


def _sc_kernel(tab_hbm, ei_hbm, adj0_hbm, adj1_hbm, out_hbm,
               acc_sh, col_v, row_v, adj0_v, adj1_v, rows_v, gsems, ssems):
    cid = lax.axis_index("c")
    sid = lax.axis_index("s")
    wid = cid * NS + sid

    zero = jnp.zeros((LANES,), jnp.float32)

    def zfill(r, carry):
        for f in range(UNITS // LANES):
            rows_v[0, r, pl.ds(f * LANES, LANES)] = zero
        return carry

    lax.fori_loop(0, C, zfill, None, unroll=4)
    rem = ROWS_PT % C
    zdescs = []
    for k in range(ROWS_PT // C):
        zdescs.append(pltpu.make_async_copy(
            rows_v.at[0], acc_sh.at[pl.ds(sid * ROWS_PT + k * C, C)],
            ssems.at[0],
        ))
    zdescs.append(pltpu.make_async_copy(
        rows_v.at[0].at[pl.ds(0, rem)],
        acc_sh.at[pl.ds(sid * ROWS_PT + (ROWS_PT // C) * C, rem)],
        ssems.at[0],
    ))
    for d in zdescs:
        d.start()
    for d in zdescs:
        d.wait()

    plsc.subcore_barrier()

    def start_gather(j, b):
        pltpu.async_copy(
            tab_hbm.at[col_v.at[pl.ds(j * C, C)]], rows_v.at[b], gsems.at[b]
        )

    def wait_gather(j, b):
        pltpu.make_async_copy(
            tab_hbm.at[col_v.at[pl.ds(j * C, C)]], rows_v.at[b], gsems.at[b]
        ).wait()

    def start_scatter(j, b):
        pltpu.async_copy(
            rows_v.at[b], acc_sh.at[row_v.at[pl.ds(j * C, C)]], ssems.at[b],
            add=True,
        )

    def wait_scatter(j, b):
        pltpu.make_async_copy(
            rows_v.at[b], acc_sh.at[row_v.at[pl.ds(j * C, C)]], ssems.at[b]
        ).wait()

    def scale_chunk(j, b):
        buf = rows_v.at[b]

        def ebody(e, carry):
            idx = jnp.full((LANES,), j * C + e, jnp.int32)
            a0 = plsc.load_gather(adj0_v, [idx])
            a1 = plsc.load_gather(adj1_v, [idx])
            for f in range(DIM // LANES):
                sl = pl.ds(f * LANES, LANES)
                buf[e, sl] = buf[e, sl] * a0
            for f in range(DIM // LANES, UNITS // LANES):
                sl = pl.ds(f * LANES, LANES)
                buf[e, sl] = buf[e, sl] * a1
            return carry

        lax.fori_loop(0, C, ebody, None, unroll=8)

    def super_body(s, carry):
        sbase = wid * EPW + s * SEDGES
        sdescs = [
            pltpu.make_async_copy(
                ei_hbm.at[1, pl.ds(sbase, SEDGES)], col_v, gsems.at[0]),
            pltpu.make_async_copy(
                ei_hbm.at[0, pl.ds(sbase, SEDGES)], row_v, gsems.at[0]),
            pltpu.make_async_copy(
                adj0_hbm.at[pl.ds(sbase, SEDGES)], adj0_v, gsems.at[0]),
            pltpu.make_async_copy(
                adj1_hbm.at[pl.ds(sbase, SEDGES)], adj1_v, gsems.at[0]),
        ]
        for d in sdescs:
            d.start()
        for d in sdescs:
            d.wait()

        for b in range(NBUF - 1):
            start_gather(b, b)

        def body(i, c2):
            for b in range(NBUF):
                j = i * NBUF + b
                bprev = (b - 1) % NBUF
                wait_gather(j, b)
                scale_chunk(j, b)
                start_scatter(j, b)

                @pl.when(j >= 1)
                def _():
                    wait_scatter(j - 1, bprev)

                @pl.when(j + NBUF - 1 < SB)
                def _():
                    start_gather(j + NBUF - 1, bprev)
            return c2

        lax.fori_loop(0, SB // NBUF, body, None)
        wait_scatter(SB - 1, (SB - 1) % NBUF)
        return carry

    lax.fori_loop(0, NSUPER, super_body, None)

    plsc.subcore_barrier()
    pltpu.sync_copy(
        acc_sh.at[pl.ds(sid * ROWS_PT, ROWS_PT)],
        out_hbm.at[cid, pl.ds(sid * ROWS_PT, ROWS_PT)],
    )


def _sc_stage(tab, ei, adj0, adj1):
    mesh = plsc.VectorSubcoreMesh(core_axis_name="c", subcore_axis_name="s")
    run = functools.partial(
        pl.kernel,
        out_type=jax.ShapeDtypeStruct((NC, N, UNITS), jnp.float32),
        mesh=mesh,
        scratch_types=[
            pltpu.VMEM_SHARED((N, UNITS), jnp.float32),
            pltpu.VMEM((SEDGES,), jnp.int32),
            pltpu.VMEM((SEDGES,), jnp.int32),
            pltpu.VMEM((SEDGES,), jnp.float32),
            pltpu.VMEM((SEDGES,), jnp.float32),
            pltpu.VMEM((NBUF, C, UNITS), jnp.float32),
            pltpu.SemaphoreType.DMA((NBUF,)),
            pltpu.SemaphoreType.DMA((NBUF,)),
        ],
        compiler_params=pltpu.CompilerParams(
            use_tc_tiling_on_sc=False, needs_layout_passes=False
        ),
    )(_sc_kernel)
    return run(tab, ei, adj0, adj1)


def kernel(features, edge_index, adj0_vals, adj1_vals, kernel):
    tab, kl = _tc_stage(features, kernel)
    partials = _sc_stage(tab, edge_index, adj0_vals, adj1_vals)
    output = _merge_stage(partials)
    return (output, kl[0, 0])


# speedup vs baseline: 3.4466x; 1.0441x over previous
import functools

import jax
import jax.numpy as jnp
from jax import lax
from jax.experimental import pallas as pl
from jax.experimental.pallas import tpu as pltpu
from jax.experimental.pallas import tpu_sc as plsc

N = 10000
E = 320000
D_FEAT = 128
UNITS = 128
DIM = UNITS // 2
GAMMA = 1.0

NC = 2
NS = 16
NW = NC * NS
LANES = 16
C = 40
EPW = E // NW
NBUF = 5
SB = 125
SEDGES = SB * C
NSUPER = EPW // SEDGES
ROWS_PT = N // NS

TC_BLOCK = 2000


def _tc_body(x_ref, w_ref, tab_ref, kl_ref):
    i = pl.program_id(0)
    h = jnp.dot(x_ref[...], w_ref[...], preferred_element_type=jnp.float32)
    m = h[:, :DIM]
    v = h[:, DIM:]
    mean = jnp.where(m > 0, m, jnp.exp(jnp.minimum(m, 0.0)) - 1.0)
    var = jnp.maximum(v, 0.0)
    att = jnp.exp(-GAMMA * var)
    tab_ref[:, :DIM] = mean * att
    tab_ref[:, DIM:] = var * (att * att)
    kl_part = 0.5 * jnp.sum(
        jnp.mean(jnp.square(mean) + var - jnp.log(1e-8 + var) - 1.0, axis=1)
    )

    @pl.when(i == 0)
    def _():
        kl_ref[0, 0] = 0.0

    kl_ref[0, 0] += kl_part


def _tc_stage(features, w):
    return pl.pallas_call(
        _tc_body,
        grid=(N // TC_BLOCK,),
        in_specs=[
            pl.BlockSpec((TC_BLOCK, D_FEAT), lambda i: (i, 0)),
            pl.BlockSpec((D_FEAT, UNITS), lambda i: (0, 0)),
        ],
        out_specs=[
            pl.BlockSpec((TC_BLOCK, UNITS), lambda i: (i, 0)),
            pl.BlockSpec(
                block_shape=(1, 1),
                index_map=lambda i: (0, 0),
                memory_space=pltpu.SMEM,
            ),
        ],
        out_shape=[
            jax.ShapeDtypeStruct((N, UNITS), jnp.float32),
            jax.ShapeDtypeStruct((1, 1), jnp.float32),
        ],
    )(features, w)


def _merge_body(p_ref, out_ref):
    out_ref[...] = p_ref[0] + p_ref[1]


def _merge_stage(partials):
    return pl.pallas_call(
        _merge_body,
        grid=(N // TC_BLOCK,),
        in_specs=[pl.BlockSpec((2, TC_BLOCK, UNITS), lambda i: (0, i, 0))],
        out_specs=pl.BlockSpec((TC_BLOCK, UNITS), lambda i: (i, 0)),
        out_shape=jax.ShapeDtypeStruct((N, UNITS), jnp.float32),
    )(partials)


def _sc_kernel(tab_hbm, ei_hbm, adj0_hbm, adj1_hbm, out_hbm,
               acc_sh, col_v, row_v, adj0_v, adj1_v, rows_v, gsems, ssems):
    cid = lax.axis_index("c")
    sid = lax.axis_index("s")
    wid = cid * NS + sid

    zero = jnp.zeros((LANES,), jnp.float32)

    def zfill(r, carry):
        for f in range(UNITS // LANES):
            rows_v[0, r, pl.ds(f * LANES, LANES)] = zero
        return carry

    lax.fori_loop(0, C, zfill, None, unroll=4)
    rem = ROWS_PT % C
    zdescs = []
    for k in range(ROWS_PT // C):
        zdescs.append(pltpu.make_async_copy(
            rows_v.at[0], acc_sh.at[pl.ds(sid * ROWS_PT + k * C, C)],
            ssems.at[0],
        ))
    zdescs.append(pltpu.make_async_copy(
        rows_v.at[0].at[pl.ds(0, rem)],
        acc_sh.at[pl.ds(sid * ROWS_PT + (ROWS_PT // C) * C, rem)],
        ssems.at[0],
    ))
    for d in zdescs:
        d.start()
    for d in zdescs:
        d.wait()

    plsc.subcore_barrier()

    def start_gather(j, b):
        pltpu.async_copy(
            tab_hbm.at[col_v.at[pl.ds(j * C, C)]], rows_v.at[b], gsems.at[b]
        )

    def wait_gather(j, b):
        pltpu.make_async_copy(
            tab_hbm.at[col_v.at[pl.ds(j * C, C)]], rows_v.at[b], gsems.at[b]
        ).wait()

    def start_scatter(j, b):
        pltpu.async_copy(
            rows_v.at[b], acc_sh.at[row_v.at[pl.ds(j * C, C)]], ssems.at[b],
            add=True,
        )

    def wait_scatter(j, b):
        pltpu.make_async_copy(
            rows_v.at[b], acc_sh.at[row_v.at[pl.ds(j * C, C)]], ssems.at[b]
        ).wait()

    def scale_chunk(j, b):
        buf = rows_v.at[b]

        def ebody(e, carry):
            idx = jnp.full((LANES,), j * C + e, jnp.int32)
            a0 = plsc.load_gather(adj0_v, [idx])
            a1 = plsc.load_gather(adj1_v, [idx])
            for f in range(DIM // LANES):
                sl = pl.ds(f * LANES, LANES)
                buf[e, sl] = buf[e, sl] * a0
            for f in range(DIM // LANES, UNITS // LANES):
                sl = pl.ds(f * LANES, LANES)
                buf[e, sl] = buf[e, sl] * a1
            return carry

        lax.fori_loop(0, C, ebody, None, unroll=8)

    def super_body(s, carry):
        sbase = wid * EPW + s * SEDGES
        sdescs = [
            pltpu.make_async_copy(
                ei_hbm.at[1, pl.ds(sbase, SEDGES)], col_v, gsems.at[0]),
            pltpu.make_async_copy(
                ei_hbm.at[0, pl.ds(sbase, SEDGES)], row_v, gsems.at[0]),
            pltpu.make_async_copy(
                adj0_hbm.at[pl.ds(sbase, SEDGES)], adj0_v, gsems.at[0]),
            pltpu.make_async_copy(
                adj1_hbm.at[pl.ds(sbase, SEDGES)], adj1_v, gsems.at[0]),
        ]
        for d in sdescs:
            d.start()
        for d in sdescs:
            d.wait()

        for b in range(NBUF - 1):
            start_gather(b, b)

        def body(i, c2):
            for b in range(NBUF):
                j = i * NBUF + b
                bprev = (b - 1) % NBUF
                wait_gather(j, b)
                scale_chunk(j, b)
                start_scatter(j, b)

                @pl.when(j >= 1)
                def _():
                    wait_scatter(j - 1, bprev)

                @pl.when(j + NBUF - 1 < SB)
                def _():
                    start_gather(j + NBUF - 1, bprev)
            return c2

        lax.fori_loop(0, SB // NBUF, body, None)
        wait_scatter(SB - 1, (SB - 1) % NBUF)
        return carry

    lax.fori_loop(0, NSUPER, super_body, None)

    plsc.subcore_barrier()
    pltpu.sync_copy(
        acc_sh.at[pl.ds(sid * ROWS_PT, ROWS_PT)],
        out_hbm.at[cid, pl.ds(sid * ROWS_PT, ROWS_PT)],
    )


def _sc_stage(tab, ei, adj0, adj1):
    mesh = plsc.VectorSubcoreMesh(core_axis_name="c", subcore_axis_name="s")
    run = functools.partial(
        pl.kernel,
        out_type=jax.ShapeDtypeStruct((NC, N, UNITS), jnp.float32),
        mesh=mesh,
        scratch_types=[
            pltpu.VMEM_SHARED((N, UNITS), jnp.float32),
            pltpu.VMEM((SEDGES,), jnp.int32),
            pltpu.VMEM((SEDGES,), jnp.int32),
            pltpu.VMEM((SEDGES,), jnp.float32),
            pltpu.VMEM((SEDGES,), jnp.float32),
            pltpu.VMEM((NBUF, C, UNITS), jnp.float32),
            pltpu.SemaphoreType.DMA((NBUF,)),
            pltpu.SemaphoreType.DMA((NBUF,)),
        ],
        compiler_params=pltpu.CompilerParams(
            use_tc_tiling_on_sc=False, needs_layout_passes=False
        ),
    )(_sc_kernel)
    return run(tab, ei, adj0, adj1)


def kernel(features, edge_index, adj0_vals, adj1_vals, kernel):
    tab, kl = _tc_stage(features, kernel)
    partials = _sc_stage(tab, edge_index, adj0_vals, adj1_vals)
    output = _merge_stage(partials)
    return (output, kl[0, 0])
